# baseline (speedup 1.0000x reference)
import jax
import jax.numpy as jnp
from jax import lax
from jax.experimental import pallas as pl
from jax.experimental.pallas import tpu as pltpu
from jax.experimental.pallas import tpu_sc as plsc

TEMP = 0.2
LAMBDA_1 = 1e-4
EPS_N = 1e-12

_NC, _NSUB = 2, 16
_NW = _NC * _NSUB
_R = 1000


def _spmm4_sc(adj_row, adj_col, adj_val, x_es, x_ej, x_gs, x_gj):
    nj, d = x_es.shape
    e = adj_row.shape[0]
    ek = 80
    epw = e // _NW
    nblk = epw // ek
    npair = (nblk - 1) // 2
    slab = (nj // (8 * _NSUB)) * 8
    tail = nj - slab * _NSUB
    nzfull = slab // ek
    zrem = slab - nzfull * ek
    nv = d // 16

    mesh = plsc.VectorSubcoreMesh(core_axis_name="c", subcore_axis_name="s")

    def body(row_h, col_h, val_h, es_h, ej_h, gs_h, gj_h, out_h,
             accum, eidx_row, eidx_col, evals,
             didx0, didx1, rows0, rows1, gsem0, gsem1, ssem0, ssem1):
        c = lax.axis_index("c")
        s = lax.axis_index("s")
        wid = s * _NC + c
        zero16 = jnp.zeros((16,), jnp.float32)

        def zero_rows0():
            def zb(i, carry):
                for j in range(nv):
                    rows0[i, pl.ds(j * 16, 16)] = zero16
                return carry
            lax.fori_loop(0, ek, zb, 0)

        slab0 = s * slab
        ebase0 = wid * epw
        pltpu.sync_copy(row_h.at[pl.ds(ebase0, epw)], eidx_row)
        pltpu.sync_copy(col_h.at[pl.ds(ebase0, epw)], eidx_col)
        pltpu.sync_copy(val_h.at[pl.ds(ebase0, epw)], evals)

        didx = (didx0, didx1)
        rows = (rows0, rows1)
        gsem = (gsem0, gsem1)
        ssem = (ssem0, ssem1)

        def scale_rows(rb, i):
            @plsc.parallel_loop(0, ek // 16, unroll=2)
            def _scale(g):
                vv = evals[pl.ds(i * ek + g * 16, 16)]
                for t in range(16):
                    v = vv[t]
                    for j in range(nv):
                        sl = pl.ds(j * 16, 16)
                        rb[g * 16 + t, sl] = rb[g * 16 + t, sl] * v

        def fill_didx(b, src_big, i):
            for j in range(ek // 16):
                didx[b][pl.ds(j * 16, 16)] = src_big[pl.ds(i * ek + j * 16, 16)]

        for oi, (src_big, dst_big, x_h) in enumerate((
                (eidx_col, eidx_row, es_h), (eidx_row, eidx_col, ej_h),
                (eidx_col, eidx_row, gs_h), (eidx_row, eidx_col, gj_h))):
            zero_rows0()
            for z in range(nzfull):
                pltpu.sync_copy(rows0, accum.at[pl.ds(slab0 + z * ek, ek)])
            pltpu.sync_copy(rows0.at[pl.ds(0, zrem)],
                            accum.at[pl.ds(slab0 + nzfull * ek, zrem)])

            @pl.when(s == _NSUB - 1)
            def _zero_tail():
                pltpu.sync_copy(rows0.at[pl.ds(0, tail)],
                                accum.at[pl.ds(_NSUB * slab, tail)])
            plsc.subcore_barrier()

            def gather_start(b, i):
                pltpu.async_copy(
                    x_h.at[src_big.at[pl.ds(i * ek, ek)]], rows[b], gsem[b])

            def gather_wait(b):
                pltpu.make_async_copy(
                    x_h.at[src_big.at[pl.ds(0, ek)]], rows[b], gsem[b]).wait()

            def scatter_start(b):
                pltpu.async_copy(rows[b], accum.at[didx[b]], ssem[b], add=True)

            def scatter_wait(b):
                pltpu.make_async_copy(
                    rows[b], accum.at[didx[b]], ssem[b]).wait()

            gather_start(0, 0)

            def pair(g, carry):
                for b in range(2):
                    i = 2 * g + b
                    gather_wait(b)
                    if b == 0:
                        @pl.when(g >= 1)
                        def _():
                            scatter_wait(1)
                    else:
                        scatter_wait(0)
                    gather_start(1 - b, i + 1)
                    scale_rows(rows[b], i)
                    fill_didx(b, dst_big, i)
                    scatter_start(b)
                return carry
            lax.fori_loop(0, npair, pair, 0)

            gather_wait(0)
            scatter_wait(1)
            scale_rows(rows0, nblk - 1)
            fill_didx(0, dst_big, nblk - 1)
            pltpu.sync_copy(rows0, accum.at[didx0], add=True)

            plsc.subcore_barrier()
            pltpu.sync_copy(accum.at[pl.ds(slab0, slab)],
                            out_h.at[oi].at[pl.ds(c * nj + slab0, slab)])

            @pl.when(s == _NSUB - 1)
            def _copy_tail():
                pltpu.sync_copy(
                    accum.at[pl.ds(_NSUB * slab, tail)],
                    out_h.at[oi].at[pl.ds(c * nj + _NSUB * slab, tail)])

    spmm4 = pl.kernel(
        body,
        out_type=jax.ShapeDtypeStruct((4, 2 * nj, d), jnp.float32),
        mesh=mesh,
        scratch_types=[
            pltpu.VMEM_SHARED((nj, d), jnp.float32),
            pltpu.VMEM((epw,), jnp.int32),
            pltpu.VMEM((epw,), jnp.int32),
            pltpu.VMEM((epw,), jnp.float32),
            pltpu.VMEM((ek,), jnp.int32),
            pltpu.VMEM((ek,), jnp.int32),
            pltpu.VMEM((ek, d), jnp.float32),
            pltpu.VMEM((ek, d), jnp.float32),
            pltpu.SemaphoreType.DMA,
            pltpu.SemaphoreType.DMA,
            pltpu.SemaphoreType.DMA,
            pltpu.SemaphoreType.DMA,
        ],
    )
    return spmm4(adj_row, adj_col, adj_val, x_es, x_ej, x_gs, x_gj)


def _gather_sc(e_jn, e_sn, g_jn, g_sn, j_ids, s_ids, negflat):
    nj, d = e_jn.shape
    b = j_ids.shape[0]
    tn = negflat.shape[0]
    bpw = b // _NW
    npw = tn // _NW
    nk = npw // 128

    mesh = plsc.VectorSubcoreMesh(core_axis_name="c", subcore_axis_name="s")

    def body(ej_h, es_h, gj_h, gs_h, jid_h, sid_h, neg_h, osel_h, oneg_h,
             idxb, rowsb, idxn, rowsn, sem):
        c = lax.axis_index("c")
        s = lax.axis_index("s")
        wid = s * _NC + c
        base = wid * bpw
        pltpu.sync_copy(jid_h.at[pl.ds(base, bpw)], idxb)
        pltpu.async_copy(gj_h.at[idxb], rowsb, sem).wait()
        pltpu.sync_copy(rowsb, osel_h.at[0].at[pl.ds(base, bpw)])
        pltpu.async_copy(ej_h.at[idxb], rowsb, sem).wait()
        pltpu.sync_copy(rowsb, osel_h.at[1].at[pl.ds(base, bpw)])
        pltpu.sync_copy(sid_h.at[pl.ds(base, bpw)], idxb)
        pltpu.async_copy(gs_h.at[idxb], rowsb, sem).wait()
        pltpu.sync_copy(rowsb, osel_h.at[2].at[pl.ds(base, bpw)])
        pltpu.async_copy(es_h.at[idxb], rowsb, sem).wait()
        pltpu.sync_copy(rowsb, osel_h.at[3].at[pl.ds(base, bpw)])
        for k in range(nk):
            nb = wid * npw + k * 128
            pltpu.sync_copy(neg_h.at[pl.ds(nb, 128)], idxn)
            pltpu.async_copy(es_h.at[idxn], rowsn, sem).wait()
            pltpu.sync_copy(rowsn, oneg_h.at[pl.ds(nb, 128)])

    g = pl.kernel(
        body,
        out_type=(jax.ShapeDtypeStruct((4, b, d), jnp.float32),
                  jax.ShapeDtypeStruct((tn, d), jnp.float32)),
        mesh=mesh,
        scratch_types=[
            pltpu.VMEM((bpw,), jnp.int32),
            pltpu.VMEM((bpw, d), jnp.float32),
            pltpu.VMEM((128,), jnp.int32),
            pltpu.VMEM((128, d), jnp.float32),
            pltpu.SemaphoreType.DMA,
        ],
    )
    return g(e_jn, e_sn, g_jn, g_sn, j_ids, s_ids, negflat)


def _norm1_tc(x):
    n, d = x.shape

    def body(x_ref, o_ref):
        v = x_ref[...]
        nn = jnp.sqrt(jnp.sum(v * v, axis=1, keepdims=True))
        o_ref[...] = v / jnp.maximum(nn, EPS_N)

    return pl.pallas_call(
        body,
        grid=(n // _R,),
        in_specs=[pl.BlockSpec((_R, d), lambda i: (i, 0))],
        out_specs=pl.BlockSpec((_R, d), lambda i: (i, 0)),
        out_shape=jax.ShapeDtypeStruct((n, d), jnp.float32),
    )(x)


def _norm_mean_tc(xs):
    n, d = xs[0].shape
    k = len(xs)

    def body(*refs):
        o_ref = refs[-1]
        v = refs[0][...]
        for r in refs[1:-1]:
            v = v + r[...]
        v = v / float(k)
        nn = jnp.sqrt(jnp.sum(v * v, axis=1, keepdims=True))
        o_ref[...] = v / jnp.maximum(nn, EPS_N)

    return pl.pallas_call(
        body,
        grid=(n // _R,),
        in_specs=[pl.BlockSpec((_R, d), lambda i: (i, 0))] * k,
        out_specs=pl.BlockSpec((_R, d), lambda i: (i, 0)),
        out_shape=jax.ShapeDtypeStruct((n, d), jnp.float32),
    )(*xs)


def _layer_tc(p2, res, w, bvec):
    n, d = res.shape

    def body(p_ref, r_ref, w_ref, b_ref, o_ref):
        x = p_ref[0] + p_ref[1]
        y = lax.dot_general(x, w_ref[...], (((1,), (1,)), ((), ())),
                            preferred_element_type=jnp.float32)
        y = y + b_ref[...]
        o_ref[...] = r_ref[...] + jnp.where(y >= 0, y, y * 0.01)

    return pl.pallas_call(
        body,
        grid=(n // _R,),
        in_specs=[
            pl.BlockSpec((2, _R, d), lambda i: (0, i, 0)),
            pl.BlockSpec((_R, d), lambda i: (i, 0)),
            pl.BlockSpec((d, d), lambda i: (0, 0)),
            pl.BlockSpec((1, d), lambda i: (0, 0)),
        ],
        out_specs=pl.BlockSpec((_R, d), lambda i: (i, 0)),
        out_shape=jax.ShapeDtypeStruct((n, d), jnp.float32),
    )(p2, res, w, bvec.reshape(1, d))


def _score_tc(sel, neg_rows, wstk, bstk):
    nneg, b, d = neg_rows.shape

    def body(sel_ref, neg_ref, w_ref, bb_ref, o1, o2, op, org):
        i = pl.program_id(0)

        @pl.when(i == 0)
        def _():
            gj = sel_ref[0]
            ej = sel_ref[1]
            gs = sel_ref[2]
            es = sel_ref[3]
            s1 = lax.dot_general(gj, ej, (((1,), (1,)), ((), ())),
                                 preferred_element_type=jnp.float32)
            t1 = jnp.sum(jnp.exp(s1 / TEMP), axis=1)
            o1[...] = jnp.sum(jnp.log(t1 + 1e-8)).reshape(1, 1)
            pj = jnp.clip(jnp.sum(gj * ej, axis=1) / TEMP, -1.0, 1.0)
            ps = jnp.clip(jnp.sum(gs * es, axis=1) / TEMP, -1.0, 1.0)
            op[...] = (jnp.sum(pj) + jnp.sum(ps)).reshape(1, 1)
            org[...] = (jnp.sum(w_ref[...] * w_ref[...]) +
                        jnp.sum(bb_ref[...] * bb_ref[...])).reshape(1, 1)
            o2[...] = jnp.zeros((1, 1), jnp.float32)

        @pl.when(i > 0)
        def _():
            gs = sel_ref[2]
            sn = lax.dot_general(gs, neg_ref[0], (((1,), (1,)), ((), ())),
                                 preferred_element_type=jnp.float32)
            tn = jnp.sum(jnp.exp(sn / TEMP), axis=1)
            o2[...] = o2[...] + jnp.sum(jnp.log(tn + 1e-8)).reshape(1, 1)

    return pl.pallas_call(
        body,
        grid=(nneg + 1,),
        in_specs=[
            pl.BlockSpec((4, b, d), lambda i: (0, 0, 0)),
            pl.BlockSpec((1, b, d), lambda i: (jnp.maximum(i - 1, 0), 0, 0)),
            pl.BlockSpec(wstk.shape, lambda i: (0, 0, 0)),
            pl.BlockSpec(bstk.shape, lambda i: (0, 0)),
        ],
        out_specs=[pl.BlockSpec((1, 1), lambda i: (0, 0))] * 4,
        out_shape=[jax.ShapeDtypeStruct((1, 1), jnp.float32)] * 4,
    )(sel, neg_rows, wstk, bstk)


def kernel(e_j, e_s, g_j0, g_s0, adj_val, Wj, bj, Ws, bs, Wja, bja, Wsa, bsa,
           adj_row, adj_col, j_ids, s_ids, negs):
    nl = Wj.shape[0]
    nj, d = e_j.shape
    b = j_ids.shape[0]
    nneg = negs.shape[0]

    gj0 = _norm1_tc(g_j0)
    gs0 = _norm1_tc(g_s0)
    Ejs, Ess, Gjs, Gss = [e_j], [e_s], [gj0], [gs0]
    for l in range(nl):
        part = _spmm4_sc(adj_row, adj_col, adj_val,
                         Ess[l], Ejs[l], Gss[l], Gjs[l])
        part = part.reshape(4, 2, nj, d)
        Ejs.append(_layer_tc(part[0], Ejs[l], Wj[l], bj[l]))
        Ess.append(_layer_tc(part[1], Ess[l], Ws[l], bs[l]))
        Gjs.append(_layer_tc(part[2], Gjs[l], Wja[l], bja[l]))
        Gss.append(_layer_tc(part[3], Gss[l], Wsa[l], bsa[l]))

    e_jn = _norm_mean_tc(Ejs)
    e_sn = _norm_mean_tc(Ess)
    g_jn = _norm_mean_tc(Gjs)
    g_sn = _norm_mean_tc(Gss)

    sel, neg_rows = _gather_sc(e_jn, e_sn, g_jn, g_sn,
                               j_ids, s_ids, negs.reshape(-1))
    neg_rows = neg_rows.reshape(nneg, b, d)
    wstk = jnp.stack([Wj, Ws, Wja, Wsa]).reshape(-1, d, d)
    bstk = jnp.stack([bj, bs, bja, bsa]).reshape(-1, d)
    o1, o2, op, org = _score_tc(sel, neg_rows, wstk, bstk)

    neg_score = o1[0, 0] / b + o2[0, 0] / (b * nneg)
    pos_score = op[0, 0] / b
    loss_cl = (-pos_score + neg_score) * 0.2
    loss_reg = org[0, 0] * LAMBDA_1
    loss = loss_cl + loss_reg
    return (loss, loss_cl, loss_reg)


# speedup vs baseline: 1.0963x; 1.0963x over previous
import jax
import jax.numpy as jnp
from jax import lax
from jax.experimental import pallas as pl
from jax.experimental.pallas import tpu as pltpu
from jax.experimental.pallas import tpu_sc as plsc

TEMP = 0.2
LAMBDA_1 = 1e-4
EPS_N = 1e-12

_NC, _NSUB = 2, 16
_NW = _NC * _NSUB
_R = 1000


def _spmm4_sc(adj_row, adj_col, adj_val, xflat, nj):
    d = xflat.shape[1]
    e = adj_row.shape[0]
    ek = 80
    epw = e // _NW
    nblk = epw // ek
    nd = 4
    slab = (nj // (8 * _NSUB)) * 8
    tail = nj - slab * _NSUB
    nzfull = slab // ek
    zrem = slab - nzfull * ek
    nv = d // 16
    ni = ek // 16

    mesh = plsc.VectorSubcoreMesh(core_axis_name="c", subcore_axis_name="s")

    def body(row_h, col_h, val_h, x_h, out_h, accum,
             rbuf0, rbuf1, rbuf2, rbuf3, cbuf0, cbuf1, cbuf2, cbuf3,
             sidx0, sidx1, sidx2, sidx3, didx0, didx1, didx2, didx3,
             vbuf0, vbuf1, vbuf2, vbuf3, rows0, rows1, rows2, rows3,
             isem0, isem1, isem2, isem3, gsem0, gsem1, gsem2, gsem3,
             ssem0, ssem1, ssem2, ssem3):
        c = lax.axis_index("c")
        s = lax.axis_index("s")
        wid = s * _NC + c
        zero16 = jnp.zeros((16,), jnp.float32)

        rbuf = (rbuf0, rbuf1, rbuf2, rbuf3)
        cbuf = (cbuf0, cbuf1, cbuf2, cbuf3)
        sidx = (sidx0, sidx1, sidx2, sidx3)
        didx = (didx0, didx1, didx2, didx3)
        vbuf = (vbuf0, vbuf1, vbuf2, vbuf3)
        rows = (rows0, rows1, rows2, rows3)
        isem = (isem0, isem1, isem2, isem3)
        gsem = (gsem0, gsem1, gsem2, gsem3)
        ssem = (ssem0, ssem1, ssem2, ssem3)

        slab0 = s * slab
        ebase0 = wid * epw

        def zero_rows0():
            @plsc.parallel_loop(0, ek, unroll=2)
            def _zb(i):
                for j in range(nv):
                    rows0[i, pl.ds(j * 16, 16)] = zero16

        def scale_rows(rb, vb):
            @plsc.parallel_loop(0, ni, unroll=2)
            def _scale(g):
                vv = vb[pl.ds(g * 16, 16)]
                for t in range(16):
                    v = vv[t]
                    for j in range(nv):
                        sl = pl.ds(j * 16, 16)
                        rb[g * 16 + t, sl] = rb[g * 16 + t, sl] * v

        def spmm(oi, carry):
            src_is_col = (oi % 2) == 0
            xoff = (oi ^ 1) * nj

            zero_rows0()
            for z in range(nzfull):
                pltpu.sync_copy(rows0, accum.at[pl.ds(slab0 + z * ek, ek)])
            pltpu.sync_copy(rows0.at[pl.ds(0, zrem)],
                            accum.at[pl.ds(slab0 + nzfull * ek, zrem)])

            @pl.when(s == _NSUB - 1)
            def _zero_tail():
                pltpu.sync_copy(rows0.at[pl.ds(0, tail)],
                                accum.at[pl.ds(_NSUB * slab, tail)])
            plsc.subcore_barrier()

            def idx_start(p, i):
                base = ebase0 + i * ek
                pltpu.async_copy(row_h.at[pl.ds(base, ek)], rbuf[p], isem[p])
                pltpu.async_copy(col_h.at[pl.ds(base, ek)], cbuf[p], isem[p])
                pltpu.async_copy(val_h.at[pl.ds(base, ek)], vbuf[p], isem[p])

            def idx_ready(p):
                pltpu.make_async_copy(
                    row_h.at[pl.ds(ebase0, ek)], rbuf[p], isem[p]).wait()
                pltpu.make_async_copy(
                    col_h.at[pl.ds(ebase0, ek)], cbuf[p], isem[p]).wait()
                pltpu.make_async_copy(
                    val_h.at[pl.ds(ebase0, ek)], vbuf[p], isem[p]).wait()
                for j in range(ni):
                    sl = pl.ds(j * 16, 16)
                    rv = rbuf[p][sl]
                    cv = cbuf[p][sl]
                    sidx[p][sl] = jnp.where(src_is_col, cv, rv) + xoff
                    didx[p][sl] = jnp.where(src_is_col, rv, cv)

            def gather_start(b):
                pltpu.async_copy(x_h.at[sidx[b]], rows[b], gsem[b])

            def gather_wait(b):
                pltpu.make_async_copy(x_h.at[sidx[b]], rows[b], gsem[b]).wait()

            def scatter_start(b):
                pltpu.async_copy(rows[b], accum.at[didx[b]], ssem[b], add=True)

            def scatter_wait(b):
                pltpu.make_async_copy(
                    rows[b], accum.at[didx[b]], ssem[b]).wait()

            idx_start(0, 0)
            idx_start(1, 1)
            idx_start(2, 2)
            idx_ready(0)
            gather_start(0)
            idx_ready(1)
            gather_start(1)

            def quad(g, carry2):
                for b in range(nd):
                    i = 4 * g + b
                    p = (b + 3) % nd

                    @pl.when(i + 3 < nblk)
                    def _prep():
                        @pl.when(i >= 1)
                        def _():
                            scatter_wait(p)
                        idx_start(p, i + 3)
                    q = (b + 2) % nd

                    @pl.when(i + 2 < nblk)
                    def _gnext():
                        idx_ready(q)
                        gather_start(q)
                    gather_wait(b)
                    scale_rows(rows[b], vbuf[b])
                    scatter_start(b)
                return carry2
            lax.fori_loop(0, nblk // nd, quad, 0)

            gather_wait(0)
            scale_rows(rows0, vbuf0)
            pltpu.sync_copy(rows0, accum.at[didx0], add=True)
            scatter_wait(1)
            scatter_wait(2)
            scatter_wait(3)

            plsc.subcore_barrier()
            obase = oi * 2 * nj + c * nj
            pltpu.sync_copy(accum.at[pl.ds(slab0, slab)],
                            out_h.at[pl.ds(obase + slab0, slab)])

            @pl.when(s == _NSUB - 1)
            def _copy_tail():
                pltpu.sync_copy(
                    accum.at[pl.ds(_NSUB * slab, tail)],
                    out_h.at[pl.ds(obase + _NSUB * slab, tail)])
            plsc.subcore_barrier()
            return carry

        lax.fori_loop(0, 4, spmm, 0)

    spmm4 = pl.kernel(
        body,
        out_type=jax.ShapeDtypeStruct((8 * nj, d), jnp.float32),
        mesh=mesh,
        scratch_types=(
            [pltpu.VMEM_SHARED((nj, d), jnp.float32)]
            + [pltpu.VMEM((ek,), jnp.int32)] * 16
            + [pltpu.VMEM((ek,), jnp.float32)] * 4
            + [pltpu.VMEM((ek, d), jnp.float32)] * 4
            + [pltpu.SemaphoreType.DMA] * 12
        ),
    )
    return spmm4(adj_row, adj_col, adj_val, xflat)


def _gather_sc(nrm, j_ids, s_ids, negflat):
    nj, d = nrm.shape[1], nrm.shape[2]
    b = j_ids.shape[0]
    tn = negflat.shape[0]
    bpw = b // _NW
    npw = tn // _NW
    nk = npw // 128

    mesh = plsc.VectorSubcoreMesh(core_axis_name="c", subcore_axis_name="s")

    def body(nrm_h, jid_h, sid_h, neg_h, osel_h, oneg_h,
             idxb, rowsb, idxn, rowsn, sem):
        c = lax.axis_index("c")
        s = lax.axis_index("s")
        wid = s * _NC + c
        base = wid * bpw
        ej_h, es_h, gj_h, gs_h = (nrm_h.at[k] for k in range(4))
        pltpu.sync_copy(jid_h.at[pl.ds(base, bpw)], idxb)
        pltpu.async_copy(gj_h.at[idxb], rowsb, sem).wait()
        pltpu.sync_copy(rowsb, osel_h.at[0].at[pl.ds(base, bpw)])
        pltpu.async_copy(ej_h.at[idxb], rowsb, sem).wait()
        pltpu.sync_copy(rowsb, osel_h.at[1].at[pl.ds(base, bpw)])
        pltpu.sync_copy(sid_h.at[pl.ds(base, bpw)], idxb)
        pltpu.async_copy(gs_h.at[idxb], rowsb, sem).wait()
        pltpu.sync_copy(rowsb, osel_h.at[2].at[pl.ds(base, bpw)])
        pltpu.async_copy(es_h.at[idxb], rowsb, sem).wait()
        pltpu.sync_copy(rowsb, osel_h.at[3].at[pl.ds(base, bpw)])
        for k in range(nk):
            nb = wid * npw + k * 128
            pltpu.sync_copy(neg_h.at[pl.ds(nb, 128)], idxn)
            pltpu.async_copy(es_h.at[idxn], rowsn, sem).wait()
            pltpu.sync_copy(rowsn, oneg_h.at[pl.ds(nb, 128)])

    g = pl.kernel(
        body,
        out_type=(jax.ShapeDtypeStruct((4, b, d), jnp.float32),
                  jax.ShapeDtypeStruct((tn, d), jnp.float32)),
        mesh=mesh,
        scratch_types=[
            pltpu.VMEM((bpw,), jnp.int32),
            pltpu.VMEM((bpw, d), jnp.float32),
            pltpu.VMEM((128,), jnp.int32),
            pltpu.VMEM((128, d), jnp.float32),
            pltpu.SemaphoreType.DMA,
        ],
    )
    return g(nrm, j_ids, s_ids, negflat)


def _prep_tc(stk):
    _, n, d = stk.shape

    def body(x_ref, o_ref):
        oi = pl.program_id(0)
        v = x_ref[0]
        nn = jnp.sqrt(jnp.sum(v * v, axis=1, keepdims=True))
        vn = v / jnp.maximum(nn, EPS_N)
        o_ref[0] = jnp.where(oi >= 2, vn, v)

    return pl.pallas_call(
        body,
        grid=(4, n // _R),
        in_specs=[pl.BlockSpec((1, _R, d), lambda oi, i: (oi, i, 0))],
        out_specs=pl.BlockSpec((1, _R, d), lambda oi, i: (oi, i, 0)),
        out_shape=jax.ShapeDtypeStruct((4, n, d), jnp.float32),
    )(stk)


def _layer4_tc(part, cur, wstk, bstk):
    _, n, d = cur.shape

    def body(p_ref, r_ref, w_ref, b_ref, o_ref):
        x = p_ref[0, 0] + p_ref[0, 1]
        y = lax.dot_general(x, w_ref[0], (((1,), (1,)), ((), ())),
                            preferred_element_type=jnp.float32)
        y = y + b_ref[0]
        o_ref[0] = r_ref[0] + jnp.where(y >= 0, y, y * 0.01)

    return pl.pallas_call(
        body,
        grid=(4, n // _R),
        in_specs=[
            pl.BlockSpec((1, 2, _R, d), lambda oi, i: (oi, 0, i, 0)),
            pl.BlockSpec((1, _R, d), lambda oi, i: (oi, i, 0)),
            pl.BlockSpec((1, d, d), lambda oi, i: (oi, 0, 0)),
            pl.BlockSpec((1, 1, d), lambda oi, i: (oi, 0, 0)),
        ],
        out_specs=pl.BlockSpec((1, _R, d), lambda oi, i: (oi, i, 0)),
        out_shape=jax.ShapeDtypeStruct((4, n, d), jnp.float32),
    )(part, cur, wstk, bstk.reshape(4, 1, d))


def _norm_mean4_tc(stks):
    n, d = stks[0].shape[1], stks[0].shape[2]
    k = len(stks)

    def body(*refs):
        o_ref = refs[-1]
        v = refs[0][0]
        for r in refs[1:-1]:
            v = v + r[0]
        v = v / float(k)
        nn = jnp.sqrt(jnp.sum(v * v, axis=1, keepdims=True))
        o_ref[0] = v / jnp.maximum(nn, EPS_N)

    return pl.pallas_call(
        body,
        grid=(4, n // _R),
        in_specs=[pl.BlockSpec((1, _R, d), lambda oi, i: (oi, i, 0))] * k,
        out_specs=pl.BlockSpec((1, _R, d), lambda oi, i: (oi, i, 0)),
        out_shape=jax.ShapeDtypeStruct((4, n, d), jnp.float32),
    )(*stks)


def _score_tc(sel, neg_rows, wstk, bstk):
    nneg, b, d = neg_rows.shape

    def body(sel_ref, neg_ref, w_ref, bb_ref, o1, o2, op, org):
        i = pl.program_id(0)

        @pl.when(i == 0)
        def _():
            gj = sel_ref[0]
            ej = sel_ref[1]
            gs = sel_ref[2]
            es = sel_ref[3]
            s1 = lax.dot_general(gj, ej, (((1,), (1,)), ((), ())),
                                 preferred_element_type=jnp.float32)
            t1 = jnp.sum(jnp.exp(s1 / TEMP), axis=1)
            o1[...] = jnp.sum(jnp.log(t1 + 1e-8)).reshape(1, 1)
            pj = jnp.clip(jnp.sum(gj * ej, axis=1) / TEMP, -1.0, 1.0)
            ps = jnp.clip(jnp.sum(gs * es, axis=1) / TEMP, -1.0, 1.0)
            op[...] = (jnp.sum(pj) + jnp.sum(ps)).reshape(1, 1)
            org[...] = (jnp.sum(w_ref[...] * w_ref[...]) +
                        jnp.sum(bb_ref[...] * bb_ref[...])).reshape(1, 1)
            o2[...] = jnp.zeros((1, 1), jnp.float32)

        @pl.when(i > 0)
        def _():
            gs = sel_ref[2]
            sn = lax.dot_general(gs, neg_ref[0], (((1,), (1,)), ((), ())),
                                 preferred_element_type=jnp.float32)
            tn = jnp.sum(jnp.exp(sn / TEMP), axis=1)
            o2[...] = o2[...] + jnp.sum(jnp.log(tn + 1e-8)).reshape(1, 1)

    return pl.pallas_call(
        body,
        grid=(nneg + 1,),
        in_specs=[
            pl.BlockSpec((4, b, d), lambda i: (0, 0, 0)),
            pl.BlockSpec((1, b, d), lambda i: (jnp.maximum(i - 1, 0), 0, 0)),
            pl.BlockSpec(wstk.shape, lambda i: (0, 0, 0)),
            pl.BlockSpec(bstk.shape, lambda i: (0, 0)),
        ],
        out_specs=[pl.BlockSpec((1, 1), lambda i: (0, 0))] * 4,
        out_shape=[jax.ShapeDtypeStruct((1, 1), jnp.float32)] * 4,
    )(sel, neg_rows, wstk, bstk)


def kernel(e_j, e_s, g_j0, g_s0, adj_val, Wj, bj, Ws, bs, Wja, bja, Wsa, bsa,
           adj_row, adj_col, j_ids, s_ids, negs):
    nl = Wj.shape[0]
    nj, d = e_j.shape
    b = j_ids.shape[0]
    nneg = negs.shape[0]

    cur = _prep_tc(jnp.stack([e_j, e_s, g_j0, g_s0]))
    layers = [cur]
    for l in range(nl):
        part = _spmm4_sc(adj_row, adj_col, adj_val,
                         cur.reshape(4 * nj, d), nj)
        wstk_l = jnp.stack([Wj[l], Ws[l], Wja[l], Wsa[l]])
        bstk_l = jnp.stack([bj[l], bs[l], bja[l], bsa[l]])
        cur = _layer4_tc(part.reshape(4, 2, nj, d), cur, wstk_l, bstk_l)
        layers.append(cur)

    nrm = _norm_mean4_tc(layers)
    sel, neg_rows = _gather_sc(nrm, j_ids, s_ids, negs.reshape(-1))
    neg_rows = neg_rows.reshape(nneg, b, d)
    wstk = jnp.stack([Wj, Ws, Wja, Wsa]).reshape(-1, d, d)
    bstk = jnp.stack([bj, bs, bja, bsa]).reshape(-1, d)
    o1, o2, op, org = _score_tc(sel, neg_rows, wstk, bstk)

    neg_score = o1[0, 0] / b + o2[0, 0] / (b * nneg)
    pos_score = op[0, 0] / b
    loss_cl = (-pos_score + neg_score) * 0.2
    loss_reg = org[0, 0] * LAMBDA_1
    loss = loss_cl + loss_reg
    return (loss, loss_cl, loss_reg)
